# Initial kernel scaffold; baseline (speedup 1.0000x reference)
import jax
import jax.numpy as jnp
from jax.experimental import pallas as pl


def kernel(src, tgt, grid, ep, maxep, H, W):
    raise NotImplementedError("write your pallas kernel here")


# speedup vs baseline: 14.5452x; 14.5452x over previous
import jax
import jax.numpy as jnp
from jax.experimental import pallas as pl
from jax.experimental.pallas import tpu as pltpu

_UP = 10
_K = 5
_STD_FACTOR = 3.0
_QB = 256
_MB = 256


def _d2mat(q, p):
    qq = jnp.sum(q * q, axis=1, keepdims=True)
    pp = jnp.sum(p * p, axis=1)
    qp = jax.lax.dot_general(q, p, (((1,), (1,)), ((), ())),
                             preferred_element_type=jnp.float32)
    return qq + pp[None, :] - 2.0 * qp


def _min_onehot(cur, iota, n):
    m = jnp.min(cur, axis=1, keepdims=True)
    tie = cur == m
    j = jnp.min(jnp.where(tie, iota, n), axis=1, keepdims=True)
    return iota == j


def _topk_select(q, pts, d2, k):
    qb, n = d2.shape
    iota = jax.lax.broadcasted_iota(jnp.int32, (qb, n), 1)
    out = []
    cur = d2
    for _ in range(k):
        onehot = _min_onehot(cur, iota, n)
        psel = jax.lax.dot_general(onehot.astype(jnp.float32), pts,
                                   (((1,), (0,)), ((), ())),
                                   preferred_element_type=jnp.float32)
        diff = q - psel
        dk = jnp.sum(diff * diff, axis=1)
        out.append((dk, psel))
        cur = jnp.where(onehot, jnp.inf, cur)
    return out


def _scalar_mask(val, shape):
    r = jax.lax.broadcasted_iota(jnp.int32, shape, 0)
    c = jax.lax.broadcasted_iota(jnp.int32, shape, 1)
    return jnp.where((r == 0) & (c == 0), val, 0.0)


def _geo_kernel(q_ref, tgt_ref, src_ref, out_ref):
    b = pl.program_id(0)
    qi = pl.program_id(1)

    @pl.when(jnp.logical_and(b == 0, qi == 0))
    def _():
        out_ref[...] = jnp.zeros_like(out_ref)

    q = q_ref[0]
    tp = tgt_ref[0]
    sp = src_ref[0]

    sel_t = _topk_select(q, tp, _d2mat(q, tp), _K)
    sel_s = _topk_select(q, sp, _d2mat(q, sp), _K)

    d1 = sel_t[0][0]
    wk = [jnp.exp(d1 - dk) for dk, _ in sel_t]
    z = wk[0]
    for w in wk[1:]:
        z = z + w
    inv_z = 1.0 / z

    udf_t = jnp.zeros_like(d1)
    gt = jnp.zeros_like(sel_t[0][1])
    for w, (dk, pk) in zip(wk, sel_t):
        udf_t = udf_t + jnp.sqrt(dk + 1e-10) * w
        gt = gt + w[:, None] * pk
    udf_s = jnp.zeros_like(d1)
    gs = jnp.zeros_like(gt)
    for w, (dk, pk) in zip(wk, sel_s):
        udf_s = udf_s + jnp.sqrt(dk + 1e-10) * w
        gs = gs + w[:, None] * pk

    udf_err = jnp.abs(udf_t - udf_s) * inv_z
    grad_err = jnp.sum(jnp.abs(gt - gs), axis=1) * inv_z
    s = jnp.sum(udf_err + grad_err)
    out_ref[...] += _scalar_mask(s, out_ref.shape)


def _std_kernel(tq_ref, tp_ref, out_ref):
    tq = tq_ref[0]
    tp = tp_ref[0]
    d2 = _d2mat(tq, tp)
    mb, n = d2.shape
    iota = jax.lax.broadcasted_iota(jnp.int32, (mb, n), 1)
    oh1 = _min_onehot(d2, iota, n)
    cur = jnp.where(oh1, jnp.inf, d2)
    oh2 = _min_onehot(cur, iota, n)
    psel = jax.lax.dot_general(oh2.astype(jnp.float32), tp,
                               (((1,), (0,)), ((), ())),
                               preferred_element_type=jnp.float32)
    diff = tq - psel
    dk = jnp.sum(diff * diff, axis=1)
    std = jnp.sqrt(dk + 1e-10) * _STD_FACTOR
    out_ref[0] = jnp.broadcast_to(std[None, :], out_ref.shape[1:])


def _smooth_kernel(img_ref, out_ref):
    img = img_ref[...]
    h, w = img.shape[2], img.shape[3]
    acc = jnp.zeros(img.shape[:2] + (h - 2, w - 2), jnp.float32)
    for i in range(3):
        for j in range(3):
            acc = acc + img[:, :, i:i + h - 2, j:j + w - 2]
    mean = acc / 9.0
    mid = img[:, :, 1:h - 1, 1:w - 1]
    val = jnp.mean(jnp.abs(mid - mean))
    out_ref[...] = _scalar_mask(val, out_ref.shape)


def kernel(src, tgt, grid, ep, maxep, H, W):
    B, N, C = src.shape
    M = tgt.shape[1]
    Q = M * _UP + N

    std_full = pl.pallas_call(
        _std_kernel,
        grid=(B, M // _MB),
        in_specs=[
            pl.BlockSpec((1, _MB, C), lambda b, m: (b, m, 0)),
            pl.BlockSpec((1, M, C), lambda b, m: (b, 0, 0)),
        ],
        out_specs=pl.BlockSpec((1, 8, _MB), lambda b, m: (b, 0, m)),
        out_shape=jax.ShapeDtypeStruct((B, 8, M), jnp.float32),
    )(tgt, tgt)
    std = std_full[:, 0, :]

    qs = []
    for b in range(B):
        kb = jax.random.fold_in(jax.random.key(42), b)
        noise = jax.random.normal(kb, (M, _UP, C), dtype=jnp.float32)
        noise = noise * std[b][:, None, None]
        qn = (tgt[b][:, None, :] + noise).reshape(-1, C)
        qs.append(jnp.concatenate([qn, src[b]], axis=0))
    query = jnp.stack(qs)

    geo_out = pl.pallas_call(
        _geo_kernel,
        grid=(B, Q // _QB),
        in_specs=[
            pl.BlockSpec((1, _QB, C), lambda b, q: (b, q, 0)),
            pl.BlockSpec((1, M, C), lambda b, q: (b, 0, 0)),
            pl.BlockSpec((1, N, C), lambda b, q: (b, 0, 0)),
        ],
        out_specs=pl.BlockSpec((8, 128), lambda b, q: (0, 0)),
        out_shape=jax.ShapeDtypeStruct((8, 128), jnp.float32),
    )(query, tgt, src)
    geo_total = geo_out[0, 0] / B / Q

    Hs = 64
    Ws = N // Hs
    src_img = jnp.transpose(src, (0, 2, 1)).reshape(B, C, Hs, Ws)
    smth_out = pl.pallas_call(
        _smooth_kernel,
        out_specs=pl.BlockSpec((8, 128), lambda: (0, 0)),
        out_shape=jax.ShapeDtypeStruct((8, 128), jnp.float32),
    )(src_img)
    smth = smth_out[0, 0]

    wsmth = (1.0 / maxep) ** 2 * (ep - maxep) ** 2
    hw_ratio = (H * W) / (Hs * Ws)
    return (geo_total + wsmth * smth) * hw_ratio


# speedup vs baseline: 18.3053x; 1.2585x over previous
import jax
import jax.numpy as jnp
from jax.experimental import pallas as pl
from jax.experimental.pallas import tpu as pltpu

_UP = 10
_K = 5
_STD_FACTOR = 3.0
_QB = 256
_MB = 256


def _d2mat(q, p):
    qq = jnp.sum(q * q, axis=1, keepdims=True)
    pp = jnp.sum(p * p, axis=1)
    qp = jax.lax.dot_general(q, p, (((1,), (1,)), ((), ())),
                             preferred_element_type=jnp.float32)
    return qq + pp[None, :] - 2.0 * qp


def _min_onehot(cur, iota, n):
    m = jnp.min(cur, axis=1, keepdims=True)
    tie = cur == m
    j = jnp.min(jnp.where(tie, iota, n), axis=1, keepdims=True)
    return iota == j


def _topk_select(q, pts, d2, k):
    qb, n = d2.shape
    iota = jax.lax.broadcasted_iota(jnp.int32, (qb, n), 1)
    out = []
    cur = d2
    for _ in range(k):
        onehot = _min_onehot(cur, iota, n)
        psel = jax.lax.dot_general(onehot.astype(jnp.float32), pts,
                                   (((1,), (0,)), ((), ())),
                                   preferred_element_type=jnp.float32)
        diff = q - psel
        dk = jnp.sum(diff * diff, axis=1)
        out.append((dk, psel))
        cur = jnp.where(onehot, jnp.inf, cur)
    return out


def _scalar_mask(val, shape):
    r = jax.lax.broadcasted_iota(jnp.int32, shape, 0)
    c = jax.lax.broadcasted_iota(jnp.int32, shape, 1)
    return jnp.where((r == 0) & (c == 0), val, 0.0)


_IDX_BITS = 12
_IDX_MASK = -(1 << _IDX_BITS)
_KEY_MAX = 2147483647


def _packed_keys(q, qq, pts):
    pp = jnp.sum(pts * pts, axis=1)
    qp = jax.lax.dot_general(q, pts, (((1,), (1,)), ((), ())),
                             preferred_element_type=jnp.float32)
    d2 = jnp.maximum(qq + pp[None, :] - 2.0 * qp, 0.0)
    iota = jax.lax.broadcasted_iota(jnp.int32, d2.shape, 1)
    return (jax.lax.bitcast_convert_type(d2, jnp.int32) & jnp.int32(_IDX_MASK)) | iota


def _geo_kernel(q_ref, tgt_ref, src_ref, out_ref):
    b = pl.program_id(0)
    qi = pl.program_id(1)

    @pl.when(jnp.logical_and(b == 0, qi == 0))
    def _():
        out_ref[...] = jnp.zeros_like(out_ref)

    q = q_ref[0]
    tp = tgt_ref[0]
    sp = src_ref[0]
    qq = jnp.sum(q * q, axis=1, keepdims=True)

    cur = _packed_keys(q, qq, tp)
    wacc_t = jnp.zeros(cur.shape, jnp.float32)
    w_list = []
    d1 = None
    udf_t = jnp.zeros_like(qq)
    for k in range(_K):
        kmin = jnp.min(cur, axis=1, keepdims=True)
        oh = cur == kmin
        dk = jax.lax.bitcast_convert_type(kmin & jnp.int32(_IDX_MASK), jnp.float32)
        if k == 0:
            d1 = dk
        w = jnp.exp(d1 - dk)
        w_list.append(w)
        udf_t = udf_t + jnp.sqrt(dk + 1e-10) * w
        wacc_t = jnp.where(oh, w, wacc_t)
        cur = jnp.where(oh, jnp.int32(_KEY_MAX), cur)

    z = w_list[0]
    for w in w_list[1:]:
        z = z + w
    inv_z = 1.0 / z

    cur = _packed_keys(q, qq, sp)
    wacc_s = jnp.zeros(cur.shape, jnp.float32)
    udf_s = jnp.zeros_like(qq)
    for k in range(_K):
        kmin = jnp.min(cur, axis=1, keepdims=True)
        oh = cur == kmin
        dk = jax.lax.bitcast_convert_type(kmin & jnp.int32(_IDX_MASK), jnp.float32)
        w = w_list[k]
        udf_s = udf_s + jnp.sqrt(dk + 1e-10) * w
        wacc_s = jnp.where(oh, w, wacc_s)
        cur = jnp.where(oh, jnp.int32(_KEY_MAX), cur)

    gdiff = (jax.lax.dot_general(wacc_t, tp, (((1,), (0,)), ((), ())),
                                 preferred_element_type=jnp.float32) -
             jax.lax.dot_general(wacc_s, sp, (((1,), (0,)), ((), ())),
                                 preferred_element_type=jnp.float32))
    err = (jnp.abs(udf_t - udf_s) +
           jnp.sum(jnp.abs(gdiff), axis=1, keepdims=True)) * inv_z
    s = jnp.sum(err)
    out_ref[...] += _scalar_mask(s, out_ref.shape)


def _std_kernel(tq_ref, tp_ref, out_ref):
    tq = tq_ref[0]
    tp = tp_ref[0]
    d2 = _d2mat(tq, tp)
    mb, n = d2.shape
    iota = jax.lax.broadcasted_iota(jnp.int32, (mb, n), 1)
    oh1 = _min_onehot(d2, iota, n)
    cur = jnp.where(oh1, jnp.inf, d2)
    oh2 = _min_onehot(cur, iota, n)
    psel = jax.lax.dot_general(oh2.astype(jnp.float32), tp,
                               (((1,), (0,)), ((), ())),
                               preferred_element_type=jnp.float32)
    diff = tq - psel
    dk = jnp.sum(diff * diff, axis=1)
    std = jnp.sqrt(dk + 1e-10) * _STD_FACTOR
    out_ref[0] = jnp.broadcast_to(std[None, :], out_ref.shape[1:])


def _smooth_kernel(img_ref, out_ref):
    img = img_ref[...]
    h, w = img.shape[2], img.shape[3]
    acc = jnp.zeros(img.shape[:2] + (h - 2, w - 2), jnp.float32)
    for i in range(3):
        for j in range(3):
            acc = acc + img[:, :, i:i + h - 2, j:j + w - 2]
    mean = acc / 9.0
    mid = img[:, :, 1:h - 1, 1:w - 1]
    val = jnp.mean(jnp.abs(mid - mean))
    out_ref[...] = _scalar_mask(val, out_ref.shape)


def kernel(src, tgt, grid, ep, maxep, H, W):
    B, N, C = src.shape
    M = tgt.shape[1]
    Q = M * _UP + N

    std_full = pl.pallas_call(
        _std_kernel,
        grid=(B, M // _MB),
        in_specs=[
            pl.BlockSpec((1, _MB, C), lambda b, m: (b, m, 0)),
            pl.BlockSpec((1, M, C), lambda b, m: (b, 0, 0)),
        ],
        out_specs=pl.BlockSpec((1, 8, _MB), lambda b, m: (b, 0, m)),
        out_shape=jax.ShapeDtypeStruct((B, 8, M), jnp.float32),
    )(tgt, tgt)
    std = std_full[:, 0, :]

    qs = []
    for b in range(B):
        kb = jax.random.fold_in(jax.random.key(42), b)
        noise = jax.random.normal(kb, (M, _UP, C), dtype=jnp.float32)
        noise = noise * std[b][:, None, None]
        qn = (tgt[b][:, None, :] + noise).reshape(-1, C)
        qs.append(jnp.concatenate([qn, src[b]], axis=0))
    query = jnp.stack(qs)

    geo_out = pl.pallas_call(
        _geo_kernel,
        grid=(B, Q // _QB),
        in_specs=[
            pl.BlockSpec((1, _QB, C), lambda b, q: (b, q, 0)),
            pl.BlockSpec((1, M, C), lambda b, q: (b, 0, 0)),
            pl.BlockSpec((1, N, C), lambda b, q: (b, 0, 0)),
        ],
        out_specs=pl.BlockSpec((8, 128), lambda b, q: (0, 0)),
        out_shape=jax.ShapeDtypeStruct((8, 128), jnp.float32),
    )(query, tgt, src)
    geo_total = geo_out[0, 0] / B / Q

    Hs = 64
    Ws = N // Hs
    src_img = jnp.transpose(src, (0, 2, 1)).reshape(B, C, Hs, Ws)
    smth_out = pl.pallas_call(
        _smooth_kernel,
        out_specs=pl.BlockSpec((8, 128), lambda: (0, 0)),
        out_shape=jax.ShapeDtypeStruct((8, 128), jnp.float32),
    )(src_img)
    smth = smth_out[0, 0]

    wsmth = (1.0 / maxep) ** 2 * (ep - maxep) ** 2
    hw_ratio = (H * W) / (Hs * Ws)
    return (geo_total + wsmth * smth) * hw_ratio


# speedup vs baseline: 19.6432x; 1.0731x over previous
import jax
import jax.numpy as jnp
from jax.experimental import pallas as pl
from jax.experimental.pallas import tpu as pltpu

_UP = 10
_K = 5
_STD_FACTOR = 3.0
_QB = 256
_MB = 256


def _d2mat(q, p):
    qq = jnp.sum(q * q, axis=1, keepdims=True)
    pp = jnp.sum(p * p, axis=1)
    qp = jax.lax.dot_general(q, p, (((1,), (1,)), ((), ())),
                             preferred_element_type=jnp.float32)
    return qq + pp[None, :] - 2.0 * qp


def _min_onehot(cur, iota, n):
    m = jnp.min(cur, axis=1, keepdims=True)
    tie = cur == m
    j = jnp.min(jnp.where(tie, iota, n), axis=1, keepdims=True)
    return iota == j


def _topk_select(q, pts, d2, k):
    qb, n = d2.shape
    iota = jax.lax.broadcasted_iota(jnp.int32, (qb, n), 1)
    out = []
    cur = d2
    for _ in range(k):
        onehot = _min_onehot(cur, iota, n)
        psel = jax.lax.dot_general(onehot.astype(jnp.float32), pts,
                                   (((1,), (0,)), ((), ())),
                                   preferred_element_type=jnp.float32)
        diff = q - psel
        dk = jnp.sum(diff * diff, axis=1)
        out.append((dk, psel))
        cur = jnp.where(onehot, jnp.inf, cur)
    return out


def _scalar_mask(val, shape):
    r = jax.lax.broadcasted_iota(jnp.int32, shape, 0)
    c = jax.lax.broadcasted_iota(jnp.int32, shape, 1)
    return jnp.where((r == 0) & (c == 0), val, 0.0)


_IDX_BITS = 12
_IDX_MASK = -(1 << _IDX_BITS)
_KEY_MAX = 2147483647


def _packed_keys(q, qq, pts):
    pp = jnp.sum(pts * pts, axis=1)
    qp = jax.lax.dot_general(q, pts, (((1,), (1,)), ((), ())),
                             preferred_element_type=jnp.float32)
    d2 = jnp.maximum(qq + pp[None, :] - 2.0 * qp, 0.0)
    iota = jax.lax.broadcasted_iota(jnp.int32, d2.shape, 1)
    return (jax.lax.bitcast_convert_type(d2, jnp.int32) & jnp.int32(_IDX_MASK)) | iota


def _top5_mins(keys):
    cur = keys
    kmins = []
    for k in range(_K):
        m = jnp.min(cur, axis=1, keepdims=True)
        kmins.append(m)
        if k < _K - 1:
            cur = jnp.where(cur == m, jnp.int32(_KEY_MAX), cur)
    return kmins


def _weight_mat(keys, kmins, ws):
    acc = jnp.zeros(keys.shape, jnp.float32)
    for m, w in zip(kmins, ws):
        acc = jnp.where(keys == m, w, acc)
    return acc


def _geo_kernel(q_ref, tgt_ref, src_ref, out_ref):
    b = pl.program_id(0)
    qi = pl.program_id(1)

    @pl.when(jnp.logical_and(b == 0, qi == 0))
    def _():
        out_ref[...] = jnp.zeros_like(out_ref)

    q = q_ref[0]
    tp = tgt_ref[0]
    sp = src_ref[0]
    qq = jnp.sum(q * q, axis=1, keepdims=True)

    keys_t = _packed_keys(q, qq, tp)
    kmins_t = _top5_mins(keys_t)
    dks_t = [jax.lax.bitcast_convert_type(m & jnp.int32(_IDX_MASK), jnp.float32)
             for m in kmins_t]
    d1 = dks_t[0]
    w_list = [jnp.exp(d1 - dk) for dk in dks_t]
    udf_t = jnp.zeros_like(qq)
    for dk, w in zip(dks_t, w_list):
        udf_t = udf_t + jnp.sqrt(dk + 1e-10) * w
    wacc_t = _weight_mat(keys_t, kmins_t, w_list)

    z = w_list[0]
    for w in w_list[1:]:
        z = z + w
    inv_z = 1.0 / z

    keys_s = _packed_keys(q, qq, sp)
    kmins_s = _top5_mins(keys_s)
    udf_s = jnp.zeros_like(qq)
    for m, w in zip(kmins_s, w_list):
        dk = jax.lax.bitcast_convert_type(m & jnp.int32(_IDX_MASK), jnp.float32)
        udf_s = udf_s + jnp.sqrt(dk + 1e-10) * w
    wacc_s = _weight_mat(keys_s, kmins_s, w_list)

    gdiff = (jax.lax.dot_general(wacc_t, tp, (((1,), (0,)), ((), ())),
                                 preferred_element_type=jnp.float32) -
             jax.lax.dot_general(wacc_s, sp, (((1,), (0,)), ((), ())),
                                 preferred_element_type=jnp.float32))
    err = (jnp.abs(udf_t - udf_s) +
           jnp.sum(jnp.abs(gdiff), axis=1, keepdims=True)) * inv_z
    s = jnp.sum(err)
    out_ref[...] += _scalar_mask(s, out_ref.shape)


def _std_kernel(tq_ref, tp_ref, out_ref):
    tq = tq_ref[0]
    tp = tp_ref[0]
    d2 = _d2mat(tq, tp)
    mb, n = d2.shape
    iota = jax.lax.broadcasted_iota(jnp.int32, (mb, n), 1)
    oh1 = _min_onehot(d2, iota, n)
    cur = jnp.where(oh1, jnp.inf, d2)
    oh2 = _min_onehot(cur, iota, n)
    psel = jax.lax.dot_general(oh2.astype(jnp.float32), tp,
                               (((1,), (0,)), ((), ())),
                               preferred_element_type=jnp.float32)
    diff = tq - psel
    dk = jnp.sum(diff * diff, axis=1)
    std = jnp.sqrt(dk + 1e-10) * _STD_FACTOR
    out_ref[0] = jnp.broadcast_to(std[None, :], out_ref.shape[1:])


def _smooth_kernel(img_ref, out_ref):
    img = img_ref[...]
    h, w = img.shape[2], img.shape[3]
    acc = jnp.zeros(img.shape[:2] + (h - 2, w - 2), jnp.float32)
    for i in range(3):
        for j in range(3):
            acc = acc + img[:, :, i:i + h - 2, j:j + w - 2]
    mean = acc / 9.0
    mid = img[:, :, 1:h - 1, 1:w - 1]
    val = jnp.mean(jnp.abs(mid - mean))
    out_ref[...] = _scalar_mask(val, out_ref.shape)


def kernel(src, tgt, grid, ep, maxep, H, W):
    B, N, C = src.shape
    M = tgt.shape[1]
    Q = M * _UP + N

    std_full = pl.pallas_call(
        _std_kernel,
        grid=(B, M // _MB),
        in_specs=[
            pl.BlockSpec((1, _MB, C), lambda b, m: (b, m, 0)),
            pl.BlockSpec((1, M, C), lambda b, m: (b, 0, 0)),
        ],
        out_specs=pl.BlockSpec((1, 8, _MB), lambda b, m: (b, 0, m)),
        out_shape=jax.ShapeDtypeStruct((B, 8, M), jnp.float32),
    )(tgt, tgt)
    std = std_full[:, 0, :]

    qs = []
    for b in range(B):
        kb = jax.random.fold_in(jax.random.key(42), b)
        noise = jax.random.normal(kb, (M, _UP, C), dtype=jnp.float32)
        noise = noise * std[b][:, None, None]
        qn = (tgt[b][:, None, :] + noise).reshape(-1, C)
        qs.append(jnp.concatenate([qn, src[b]], axis=0))
    query = jnp.stack(qs)

    geo_out = pl.pallas_call(
        _geo_kernel,
        grid=(B, Q // _QB),
        in_specs=[
            pl.BlockSpec((1, _QB, C), lambda b, q: (b, q, 0)),
            pl.BlockSpec((1, M, C), lambda b, q: (b, 0, 0)),
            pl.BlockSpec((1, N, C), lambda b, q: (b, 0, 0)),
        ],
        out_specs=pl.BlockSpec((8, 128), lambda b, q: (0, 0)),
        out_shape=jax.ShapeDtypeStruct((8, 128), jnp.float32),
    )(query, tgt, src)
    geo_total = geo_out[0, 0] / B / Q

    Hs = 64
    Ws = N // Hs
    src_img = jnp.transpose(src, (0, 2, 1)).reshape(B, C, Hs, Ws)
    smth_out = pl.pallas_call(
        _smooth_kernel,
        out_specs=pl.BlockSpec((8, 128), lambda: (0, 0)),
        out_shape=jax.ShapeDtypeStruct((8, 128), jnp.float32),
    )(src_img)
    smth = smth_out[0, 0]

    wsmth = (1.0 / maxep) ** 2 * (ep - maxep) ** 2
    hw_ratio = (H * W) / (Hs * Ws)
    return (geo_total + wsmth * smth) * hw_ratio


# speedup vs baseline: 23.3812x; 1.1903x over previous
import jax
import jax.numpy as jnp
from jax.experimental import pallas as pl
from jax.experimental.pallas import tpu as pltpu

_UP = 10
_K = 5
_STD_FACTOR = 3.0
_QB = 256
_MB = 256


def _d2mat(q, p):
    qq = jnp.sum(q * q, axis=1, keepdims=True)
    pp = jnp.sum(p * p, axis=1)
    qp = jax.lax.dot_general(q, p, (((1,), (1,)), ((), ())),
                             preferred_element_type=jnp.float32)
    return qq + pp[None, :] - 2.0 * qp


def _min_onehot(cur, iota, n):
    m = jnp.min(cur, axis=1, keepdims=True)
    tie = cur == m
    j = jnp.min(jnp.where(tie, iota, n), axis=1, keepdims=True)
    return iota == j


def _topk_select(q, pts, d2, k):
    qb, n = d2.shape
    iota = jax.lax.broadcasted_iota(jnp.int32, (qb, n), 1)
    out = []
    cur = d2
    for _ in range(k):
        onehot = _min_onehot(cur, iota, n)
        psel = jax.lax.dot_general(onehot.astype(jnp.float32), pts,
                                   (((1,), (0,)), ((), ())),
                                   preferred_element_type=jnp.float32)
        diff = q - psel
        dk = jnp.sum(diff * diff, axis=1)
        out.append((dk, psel))
        cur = jnp.where(onehot, jnp.inf, cur)
    return out


def _scalar_mask(val, shape):
    r = jax.lax.broadcasted_iota(jnp.int32, shape, 0)
    c = jax.lax.broadcasted_iota(jnp.int32, shape, 1)
    return jnp.where((r == 0) & (c == 0), val, 0.0)


_IDX_BITS = 12
_IDX_MASK = -(1 << _IDX_BITS)
_KEY_MAX = 2147483647


def _packed_keys(q, qq, pts):
    pp = jnp.sum(pts * pts, axis=1)
    qp2 = jax.lax.dot_general(q, pts + pts, (((1,), (1,)), ((), ())),
                              preferred_element_type=jnp.float32)
    d2 = jnp.maximum(qq + pp[None, :] - qp2, 0.0)
    iota = jax.lax.broadcasted_iota(jnp.int32, d2.shape, 1)
    packed = (jax.lax.bitcast_convert_type(d2, jnp.int32)
              & jnp.int32(_IDX_MASK)) | iota
    return jax.lax.bitcast_convert_type(packed, jnp.float32)


def _key_value(m):
    mi = jax.lax.bitcast_convert_type(m, jnp.int32) & jnp.int32(_IDX_MASK)
    return jax.lax.bitcast_convert_type(mi, jnp.float32)


def _top5_mins(keys):
    kmins = [jnp.min(keys, axis=1, keepdims=True)]
    for _ in range(_K - 1):
        flt = jnp.where(keys > kmins[-1], keys, jnp.float32(jnp.inf))
        kmins.append(jnp.min(flt, axis=1, keepdims=True))
    return kmins


def _weight_mat(keys, kmins, ws):
    acc = jnp.zeros(keys.shape, jnp.float32)
    for m, w in zip(kmins, ws):
        acc = jnp.where(keys == m, w, acc)
    return acc


def _geo_kernel(q_ref, tgt_ref, src_ref, out_ref):
    b = pl.program_id(0)
    qi = pl.program_id(1)

    @pl.when(jnp.logical_and(b == 0, qi == 0))
    def _():
        out_ref[...] = jnp.zeros_like(out_ref)

    q = q_ref[0]
    tp = tgt_ref[0]
    sp = src_ref[0]
    qq = jnp.sum(q * q, axis=1, keepdims=True)

    keys_t = _packed_keys(q, qq, tp)
    kmins_t = _top5_mins(keys_t)
    dks_t = [_key_value(m) for m in kmins_t]
    d1 = dks_t[0]
    w_list = [jnp.exp(d1 - dk) for dk in dks_t]
    udf_t = jnp.zeros_like(qq)
    for dk, w in zip(dks_t, w_list):
        udf_t = udf_t + jnp.sqrt(dk + 1e-10) * w
    wacc_t = _weight_mat(keys_t, kmins_t, w_list)

    z = w_list[0]
    for w in w_list[1:]:
        z = z + w
    inv_z = 1.0 / z

    keys_s = _packed_keys(q, qq, sp)
    kmins_s = _top5_mins(keys_s)
    udf_s = jnp.zeros_like(qq)
    for m, w in zip(kmins_s, w_list):
        udf_s = udf_s + jnp.sqrt(_key_value(m) + 1e-10) * w
    wacc_s = _weight_mat(keys_s, kmins_s, w_list)

    gdiff = (jax.lax.dot_general(wacc_t, tp, (((1,), (0,)), ((), ())),
                                 preferred_element_type=jnp.float32) -
             jax.lax.dot_general(wacc_s, sp, (((1,), (0,)), ((), ())),
                                 preferred_element_type=jnp.float32))
    err = (jnp.abs(udf_t - udf_s) +
           jnp.sum(jnp.abs(gdiff), axis=1, keepdims=True)) * inv_z
    s = jnp.sum(err)
    out_ref[...] += _scalar_mask(s, out_ref.shape)


def _std_kernel(tq_ref, tp_ref, out_ref):
    tq = tq_ref[0]
    tp = tp_ref[0]
    d2 = _d2mat(tq, tp)
    mb, n = d2.shape
    iota = jax.lax.broadcasted_iota(jnp.int32, (mb, n), 1)
    oh1 = _min_onehot(d2, iota, n)
    cur = jnp.where(oh1, jnp.inf, d2)
    oh2 = _min_onehot(cur, iota, n)
    psel = jax.lax.dot_general(oh2.astype(jnp.float32), tp,
                               (((1,), (0,)), ((), ())),
                               preferred_element_type=jnp.float32)
    diff = tq - psel
    dk = jnp.sum(diff * diff, axis=1)
    std = jnp.sqrt(dk + 1e-10) * _STD_FACTOR
    out_ref[0] = jnp.broadcast_to(std[None, :], out_ref.shape[1:])


def _smooth_kernel(img_ref, out_ref):
    img = img_ref[...]
    h, w = img.shape[2], img.shape[3]
    acc = jnp.zeros(img.shape[:2] + (h - 2, w - 2), jnp.float32)
    for i in range(3):
        for j in range(3):
            acc = acc + img[:, :, i:i + h - 2, j:j + w - 2]
    mean = acc / 9.0
    mid = img[:, :, 1:h - 1, 1:w - 1]
    val = jnp.mean(jnp.abs(mid - mean))
    out_ref[...] = _scalar_mask(val, out_ref.shape)


def kernel(src, tgt, grid, ep, maxep, H, W):
    B, N, C = src.shape
    M = tgt.shape[1]
    Q = M * _UP + N

    std_full = pl.pallas_call(
        _std_kernel,
        grid=(B, M // _MB),
        in_specs=[
            pl.BlockSpec((1, _MB, C), lambda b, m: (b, m, 0)),
            pl.BlockSpec((1, M, C), lambda b, m: (b, 0, 0)),
        ],
        out_specs=pl.BlockSpec((1, 8, _MB), lambda b, m: (b, 0, m)),
        out_shape=jax.ShapeDtypeStruct((B, 8, M), jnp.float32),
    )(tgt, tgt)
    std = std_full[:, 0, :]

    qs = []
    for b in range(B):
        kb = jax.random.fold_in(jax.random.key(42), b)
        noise = jax.random.normal(kb, (M, _UP, C), dtype=jnp.float32)
        noise = noise * std[b][:, None, None]
        qn = (tgt[b][:, None, :] + noise).reshape(-1, C)
        qs.append(jnp.concatenate([qn, src[b]], axis=0))
    query = jnp.stack(qs)

    geo_out = pl.pallas_call(
        _geo_kernel,
        grid=(B, Q // _QB),
        in_specs=[
            pl.BlockSpec((1, _QB, C), lambda b, q: (b, q, 0)),
            pl.BlockSpec((1, M, C), lambda b, q: (b, 0, 0)),
            pl.BlockSpec((1, N, C), lambda b, q: (b, 0, 0)),
        ],
        out_specs=pl.BlockSpec((8, 128), lambda b, q: (0, 0)),
        out_shape=jax.ShapeDtypeStruct((8, 128), jnp.float32),
    )(query, tgt, src)
    geo_total = geo_out[0, 0] / B / Q

    Hs = 64
    Ws = N // Hs
    src_img = jnp.transpose(src, (0, 2, 1)).reshape(B, C, Hs, Ws)
    smth_out = pl.pallas_call(
        _smooth_kernel,
        out_specs=pl.BlockSpec((8, 128), lambda: (0, 0)),
        out_shape=jax.ShapeDtypeStruct((8, 128), jnp.float32),
    )(src_img)
    smth = smth_out[0, 0]

    wsmth = (1.0 / maxep) ** 2 * (ep - maxep) ** 2
    hw_ratio = (H * W) / (Hs * Ws)
    return (geo_total + wsmth * smth) * hw_ratio
